# baseline (speedup 1.0000x reference)
import functools

import jax
import jax.numpy as jnp
import numpy as np
from jax import lax
from jax.experimental import pallas as pl
from jax.experimental.pallas import tpu as pltpu
from jax.experimental.pallas import tpu_sc as plsc

VOCAB = 1000000
D = 64
SEQ = 2048
BATCH = 4
N = SEQ * BATCH

NC = 2
NS = 16
NW = NC * NS
TPW = N // NW
SPW = SEQ // (NW // BATCH)
LANES = 128
RING = 10
IPAD = TPW + 16


def _pos_enc_np(max_len, d_model):
    position = np.arange(max_len, dtype=np.float32)[:, None]
    div_term = np.exp(
        np.arange(0, d_model, 2, dtype=np.float32) * -(np.log(10000.0) / d_model))
    pe = np.zeros((max_len, d_model), dtype=np.float32)
    pe[:, 0::2] = np.sin(position * div_term)
    pe[:, 1::2] = np.cos(position * div_term)
    return pe


_PE_T = np.ascontiguousarray(_pos_enc_np(SEQ, D).T)[None]


@functools.cache
def _make_gather():
    mesh = plsc.VectorSubcoreMesh(core_axis_name="c", subcore_axis_name="s")

    @functools.partial(
        pl.kernel,
        mesh=mesh,
        out_type=jax.ShapeDtypeStruct((BATCH, D, SEQ), jnp.float32),
        scratch_types=[
            pltpu.VMEM((IPAD,), jnp.int32),
            pltpu.VMEM((IPAD,), jnp.int32),
            pltpu.VMEM((RING, D, LANES), jnp.float32),
            pltpu.VMEM((D, TPW), jnp.float32),
            pltpu.SemaphoreType.DMA,
        ],
        compiler_params=pltpu.CompilerParams(needs_layout_passes=False),
    )
    def gather_k(tab_t, q_hbm, m_hbm, out3, q_v, m_v, ring, trows, sem):
        wid = lax.axis_index("s") * NC + lax.axis_index("c")
        base = wid * TPW
        pltpu.sync_copy(q_hbm.at[pl.ds(base, TPW)], q_v.at[pl.ds(0, TPW)])
        pltpu.sync_copy(m_hbm.at[pl.ds(base, TPW)], m_v.at[pl.ds(0, TPW)])

        def fetch(t, slot):
            qv = q_v[pl.ds(t, 16)][0]
            off = pl.multiple_of(qv * LANES, LANES)
            pltpu.async_copy(
                tab_t.at[:, pl.ds(off, LANES)], ring.at[slot], sem)

        for r in range(RING):
            fetch(r, r)

        lanes16 = lax.iota(jnp.int32, 16)

        def body(t, carry):
            slot = lax.rem(t, RING)
            pltpu.make_async_copy(
                tab_t.at[:, pl.ds(0, LANES)], ring.at[slot], sem).wait()
            m = m_v[pl.ds(t, 16)][0]
            midx = jnp.full((16,), m, jnp.int32)
            sidx = jnp.full((16,), slot, jnp.int32)
            tidx = jnp.full((16,), t, jnp.int32)
            for k in range(D // 16):
                g = plsc.load_gather(ring, [sidx, lanes16 + 16 * k, midx])
                plsc.store_scatter(trows, [lanes16 + 16 * k, tidx], g)
            tn = t + RING

            @pl.when(tn < TPW)
            def _():
                fetch(tn, lax.rem(tn, RING))

            return carry

        lax.fori_loop(0, TPW, body, 0)
        b = wid // (NW // BATCH)
        s0 = lax.rem(wid, NW // BATCH) * SPW
        pltpu.sync_copy(trows, out3.at[b].at[:, pl.ds(s0, SPW)])

    return gather_k


def _ln_body(emb_ref, pe_ref, scale_ref, bias_ref, out_ref):
    e = emb_ref[...] + pe_ref[...]
    mu = jnp.mean(e, axis=1, keepdims=True)
    d = e - mu
    var = jnp.mean(d * d, axis=1, keepdims=True)
    inv = lax.rsqrt(var + 1e-10)
    out_ref[...] = d * inv * scale_ref[...] + bias_ref[...]


def _ln(emb3, pe3, scale, bias):
    blk = 1024
    return pl.pallas_call(
        _ln_body,
        grid=(BATCH, SEQ // blk),
        in_specs=[
            pl.BlockSpec((1, D, blk), lambda b, j: (b, 0, j)),
            pl.BlockSpec((1, D, blk), lambda b, j: (0, 0, j)),
            pl.BlockSpec((1, D, 1), lambda b, j: (0, 0, 0)),
            pl.BlockSpec((1, D, 1), lambda b, j: (0, 0, 0)),
        ],
        out_specs=pl.BlockSpec((1, D, blk), lambda b, j: (b, 0, j)),
        out_shape=jax.ShapeDtypeStruct((BATCH, D, SEQ), jnp.float32),
    )(emb3, pe3, scale, bias)


def kernel(x, table, ln_scale, ln_bias):
    idx = x.T.reshape(-1).astype(jnp.int32)
    q = idx // LANES
    m = idx % LANES
    table_t = table.T
    emb3 = _make_gather()(table_t, q, m)
    pe3 = jnp.asarray(_PE_T)
    out3 = _ln(emb3, pe3, ln_scale.reshape(1, D, 1), ln_bias.reshape(1, D, 1))
    return out3.transpose(2, 0, 1)


# speedup vs baseline: 1.0496x; 1.0496x over previous
import functools

import jax
import jax.numpy as jnp
import numpy as np
from jax import lax
from jax.experimental import pallas as pl
from jax.experimental.pallas import tpu as pltpu
from jax.experimental.pallas import tpu_sc as plsc

VOCAB = 1000000
D = 64
SEQ = 2048
BATCH = 4
N = SEQ * BATCH

NC = 2
NS = 16
NW = NC * NS
TPW = N // NW
SPW = SEQ // (NW // BATCH)
LANES = 128
RING = 6
IPAD = TPW + 16
NK = D // 16


def _pos_enc_np(max_len, d_model):
    position = np.arange(max_len, dtype=np.float32)[:, None]
    div_term = np.exp(
        np.arange(0, d_model, 2, dtype=np.float32) * -(np.log(10000.0) / d_model))
    pe = np.zeros((max_len, d_model), dtype=np.float32)
    pe[:, 0::2] = np.sin(position * div_term)
    pe[:, 1::2] = np.cos(position * div_term)
    return pe


_PE_T = np.ascontiguousarray(_pos_enc_np(SEQ, D).T)


@functools.cache
def _make_fused():
    mesh = plsc.VectorSubcoreMesh(core_axis_name="c", subcore_axis_name="s")

    @functools.partial(
        pl.kernel,
        mesh=mesh,
        out_type=jax.ShapeDtypeStruct((BATCH, D, SEQ), jnp.float32),
        scratch_types=[
            pltpu.VMEM((IPAD,), jnp.int32),
            pltpu.VMEM((IPAD,), jnp.int32),
            pltpu.VMEM((RING, D, LANES), jnp.float32),
            pltpu.VMEM((D, TPW), jnp.float32),
            pltpu.VMEM((D, SPW), jnp.float32),
            pltpu.VMEM((D,), jnp.float32),
            pltpu.VMEM((D,), jnp.float32),
            pltpu.SemaphoreType.DMA,
        ],
        compiler_params=pltpu.CompilerParams(needs_layout_passes=False),
    )
    def fused_k(tab_t, q_hbm, m_hbm, pe_hbm, sc_hbm, bi_hbm, out3,
                q_v, m_v, ring, trows, pe_v, sc_v, bi_v, sem):
        wid = lax.axis_index("s") * NC + lax.axis_index("c")
        base = wid * TPW
        b = wid // (NW // BATCH)
        s0 = lax.rem(wid, NW // BATCH) * SPW
        pltpu.sync_copy(q_hbm.at[pl.ds(base, TPW)], q_v.at[pl.ds(0, TPW)])
        pltpu.sync_copy(m_hbm.at[pl.ds(base, TPW)], m_v.at[pl.ds(0, TPW)])
        pltpu.sync_copy(pe_hbm.at[:, pl.ds(s0, SPW)], pe_v)
        pltpu.sync_copy(sc_hbm, sc_v)
        pltpu.sync_copy(bi_hbm, bi_v)

        def fetch(t, slot):
            qv = q_v[pl.ds(t, 16)][0]
            off = pl.multiple_of(qv * LANES, LANES)
            pltpu.async_copy(
                tab_t.at[:, pl.ds(off, LANES)], ring.at[slot], sem)

        for r in range(RING):
            fetch(r, r)

        lanes16 = lax.iota(jnp.int32, 16)
        scale = [sc_v[pl.ds(16 * k, 16)] for k in range(NK)]
        bias = [bi_v[pl.ds(16 * k, 16)] for k in range(NK)]

        def body(t, carry):
            slot = lax.rem(t, RING)
            pltpu.make_async_copy(
                tab_t.at[:, pl.ds(0, LANES)], ring.at[slot], sem).wait()
            m = m_v[pl.ds(t, 16)][0]
            midx = jnp.full((16,), m, jnp.int32)
            sidx = jnp.full((16,), slot, jnp.int32)
            tidx = jnp.full((16,), t, jnp.int32)
            e = []
            for k in range(NK):
                g = plsc.load_gather(ring, [sidx, lanes16 + 16 * k, midx])
                p = plsc.load_gather(pe_v, [lanes16 + 16 * k, tidx])
                e.append(g + p)
            tot = jnp.sum(e[0] + e[1] + e[2] + e[3])
            mu = jnp.full((16,), tot * (1.0 / D), jnp.float32)
            d = [ek - mu for ek in e]
            sq = d[0] * d[0] + d[1] * d[1] + d[2] * d[2] + d[3] * d[3]
            xv = jnp.full((16,), jnp.sum(sq) * (1.0 / D) + 1e-10, jnp.float32)
            i = plsc.bitcast(xv, jnp.int32)
            y = plsc.bitcast(jnp.int32(0x5F3759DF) - (i >> 1), jnp.float32)
            half = xv * 0.5
            for _ in range(3):
                y = y * (1.5 - half * y * y)
            for k in range(NK):
                o = d[k] * y * scale[k] + bias[k]
                plsc.store_scatter(trows, [lanes16 + 16 * k, tidx], o)
            tn = t + RING

            @pl.when(tn < TPW)
            def _():
                fetch(tn, lax.rem(tn, RING))

            return carry

        lax.fori_loop(0, TPW, body, 0)
        pltpu.sync_copy(trows, out3.at[b].at[:, pl.ds(s0, SPW)])

    return fused_k


def kernel(x, table, ln_scale, ln_bias):
    idx = x.T.reshape(-1).astype(jnp.int32)
    q = idx // LANES
    m = idx % LANES
    table_t = table.T
    pe_t = jnp.asarray(_PE_T)
    out3 = _make_fused()(table_t, q, m, pe_t,
                         ln_scale.astype(jnp.float32),
                         ln_bias.astype(jnp.float32))
    return out3.transpose(2, 0, 1)
